# baseline (speedup 1.0000x reference)
import jax
import jax.numpy as jnp
from jax import lax
from jax.experimental import pallas as pl
from jax.experimental.pallas import tpu as pltpu
from jax.experimental.pallas import tpu_sc as plsc

N_CLS = 100000
CTX_DIM = 512
B = 1024
NC, NS = 2, 16
NW = NC * NS
BPW = B // NW
CHUNK = 4
NCH = BPW // CHUNK
PADW = 2 * BPW


def _sc_body(ci_hbm, gi_hbm, hi_hbm, sgl_hbm, all_hbm, cls_hbm, gph_hbm,
             out_hbm, all_v, ci_v, gi_v, hi_v, sgl_v, gph_v, cls_v,
             sem_g0, sem_g1, sem_o0, sem_o1, sem_a):
    wid = lax.axis_index("s") * NC + lax.axis_index("c")
    base = wid * BPW
    sem_g = (sem_g0, sem_g1)
    sem_o = (sem_o0, sem_o1)

    pltpu.sync_copy(all_hbm.at[0], all_v)
    pltpu.sync_copy(ci_hbm.at[pl.ds(wid * PADW, PADW)], ci_v)
    pltpu.sync_copy(gi_hbm.at[pl.ds(wid * PADW, PADW)], gi_v)
    pltpu.sync_copy(hi_hbm.at[pl.ds(wid * PADW, PADW)], hi_v)

    def fire_gathers(j, bf):
        sl = pl.ds(j * 8, CHUNK)
        return [
            pltpu.async_copy(sgl_hbm.at[ci_v.at[sl]], sgl_v.at[bf],
                             sem_g[bf]),
            pltpu.async_copy(cls_hbm.at[gi_v.at[sl]], cls_v.at[bf],
                             sem_g[bf]),
            pltpu.async_copy(gph_hbm.at[hi_v.at[sl]], gph_v.at[bf],
                             sem_g[bf]),
        ]

    def fire_outs(j, bf):
        cps = []
        for e in range(CHUNK):
            r = base + j * CHUNK + e
            cps.append(pltpu.async_copy(
                all_v, out_hbm.at[r, pl.ds(0, 16), :], sem_a))
            cps.append(pltpu.async_copy(
                cls_v.at[bf, e], out_hbm.at[r, pl.ds(16, 8), :], sem_o[bf]))
            cps.append(pltpu.async_copy(
                gph_v.at[bf, e], out_hbm.at[r, pl.ds(24, 4), :], sem_o[bf]))
            cps.append(pltpu.async_copy(
                sgl_v.at[bf, e], out_hbm.at[r, pl.ds(28, 4), :], sem_o[bf]))
        return cps

    gd = {0: fire_gathers(0, 0)}
    outs = {0: [], 1: []}
    for j in range(NCH):
        bf = j % 2
        for d in gd[j]:
            d.wait()
        if j + 1 < NCH:
            nb = (j + 1) % 2
            for d in outs[nb]:
                d.wait()
            outs[nb] = []
            gd[j + 1] = fire_gathers(j + 1, nb)
        outs[bf] = fire_outs(j, bf)
    for d in outs[0] + outs[1]:
        d.wait()


def kernel(cls_idx, cls_group_idx, graph_group_idx, ctx_single, ctx_all,
           ctx_cls, ctx_graph):
    mesh = plsc.VectorSubcoreMesh(core_axis_name="c", subcore_axis_name="s",
                                  num_cores=NC, num_subcores=NS)

    def repack(x):
        return jnp.repeat(x.reshape(-1, CHUNK), 2, axis=0).reshape(-1)

    run = pl.kernel(
        _sc_body,
        out_type=jax.ShapeDtypeStruct((B, 32, CTX_DIM), jnp.float32),
        mesh=mesh,
        compiler_params=pltpu.CompilerParams(use_tc_tiling_on_sc=True),
        scratch_types=[
            pltpu.VMEM((16, CTX_DIM), jnp.float32),
            pltpu.VMEM((PADW,), jnp.int32),
            pltpu.VMEM((PADW,), jnp.int32),
            pltpu.VMEM((PADW,), jnp.int32),
            pltpu.VMEM((2, CHUNK, 4, CTX_DIM), jnp.float32),
            pltpu.VMEM((2, CHUNK, 4, CTX_DIM), jnp.float32),
            pltpu.VMEM((2, CHUNK, 8, CTX_DIM), jnp.float32),
            pltpu.SemaphoreType.DMA,
            pltpu.SemaphoreType.DMA,
            pltpu.SemaphoreType.DMA,
            pltpu.SemaphoreType.DMA,
            pltpu.SemaphoreType.DMA,
        ],
    )
    return run(repack(cls_idx), repack(cls_group_idx),
               repack(graph_group_idx), ctx_single, ctx_all, ctx_cls,
               ctx_graph)
